# baseline (speedup 1.0000x reference)
import functools
import math

import jax
import jax.numpy as jnp
from jax import lax
from jax.experimental import pallas as pl
from jax.experimental.pallas import tpu as pltpu
from jax.experimental.pallas import tpu_sc as plsc

_D = 128
_LANES = 16
_SCALE = math.sqrt(float(_D))
_NC = 2
_NS = 16
_NW = _NC * _NS
_NBUF = 6
_SKEW = 3


@jax.jit
def _embed(xt, table):
    seq, nb = xt.shape
    bpw = nb // _NW
    mesh = plsc.VectorSubcoreMesh(
        core_axis_name="c", subcore_axis_name="s",
        num_cores=_NC, num_subcores=_NS)

    @functools.partial(
        pl.kernel,
        mesh=mesh,
        out_type=jax.ShapeDtypeStruct((seq, nb, _D), jnp.float32),
        scratch_types=(
            [pltpu.VMEM((seq, bpw), jnp.int32),
             pltpu.VMEM((_NBUF, bpw, _D), jnp.float32)]
            + [pltpu.SemaphoreType.DMA] * (2 * _NBUF)
        ),
    )
    def body(xt_hbm, tab_hbm, out_hbm, idx_v, bufs, *sems):
        gsems = sems[:_NBUF]
        osems = sems[_NBUF:]
        wid = lax.axis_index("s") * _NC + lax.axis_index("c")
        col0 = wid * bpw

        def start_gather(j, b):
            pltpu.async_copy(tab_hbm.at[idx_v.at[j]], bufs.at[b], gsems[b])

        def wait_gather(b):
            pltpu.make_async_copy(
                tab_hbm.at[idx_v.at[0]], bufs.at[b], gsems[b]).wait()

        def start_out(j, b):
            pltpu.async_copy(
                bufs.at[b], out_hbm.at[j, pl.ds(col0, bpw)], osems[b])

        def wait_out(b):
            pltpu.make_async_copy(
                bufs.at[b], out_hbm.at[0, pl.ds(col0, bpw)], osems[b]).wait()

        def scale(b):
            buf = bufs.at[b]

            def row(i, c):
                for k in range(_D // _LANES):
                    sl = pl.ds(k * _LANES, _LANES)
                    buf[i, sl] = buf[i, sl] * _SCALE
                return c

            lax.fori_loop(0, bpw, row, 0)

        pltpu.sync_copy(xt_hbm.at[:, pl.ds(col0, bpw)], idx_v)
        for j in range(min(_SKEW, seq)):
            start_gather(j, j)

        def visit(j, b, static):
            b2 = (b + _SKEW) % _NBUF
            if static:
                if j >= _SKEW:
                    wait_out(b2)
                if j + _SKEW < seq:
                    start_gather(j + _SKEW, b2)
            else:
                @pl.when(j >= _SKEW)
                def _():
                    wait_out(b2)

                @pl.when(j + _SKEW < seq)
                def _():
                    start_gather(j + _SKEW, b2)

            wait_gather(b)
            scale(b)
            start_out(j, b)

        n_groups = seq // _NBUF

        def group(p, carry):
            for b in range(_NBUF):
                visit(p * _NBUF + b, b, False)
            return carry

        lax.fori_loop(0, n_groups, group, 0)

        for j in range(n_groups * _NBUF, seq):
            visit(j, j % _NBUF, True)

        for t in range(min(_SKEW, seq)):
            wait_out((seq - min(_SKEW, seq) + t) % _NBUF)

    return body(xt, table)


def kernel(x, table):
    b, s = x.shape
    x = x.astype(jnp.int32)
    padded = -(-b // _NW) * _NW
    if padded != b:
        x = jnp.concatenate([x, jnp.zeros((padded - b, s), jnp.int32)])
    out_t = _embed(jnp.swapaxes(x, 0, 1), table)
    out = jnp.swapaxes(out_t, 0, 1)
    return out[:b] if padded != b else out
